# baseline (speedup 1.0000x reference)
import jax
import jax.numpy as jnp
from jax.experimental import pallas as pl


def _h_kernel(x_ref, w_ref, b_ref, h_ref):
    h = jnp.dot(x_ref[...], w_ref[...], preferred_element_type=jnp.float32)
    h_ref[...] = (h + b_ref[...]).astype(jnp.bfloat16)


def _agg_kernel(*refs):
    *adj_refs, h_ref, out_ref = refs
    nsub = len(adj_refs)
    h = h_ref[...]
    sub = out_ref.shape[0] // nsub
    for c, a_ref in enumerate(adj_refs):
        a = a_ref[...].astype(jnp.bfloat16)
        acc = jnp.dot(a, h, preferred_element_type=jnp.float32)
        out_ref[c * sub:(c + 1) * sub, :] = jnp.maximum(acc, 0.0)


def kernel(x, adj, W, b):
    n, f_in = x.shape
    f_out = W.shape[1]

    bm_h = 2000
    h = pl.pallas_call(
        _h_kernel,
        grid=(n // bm_h,),
        in_specs=[
            pl.BlockSpec((bm_h, f_in), lambda i: (i, 0)),
            pl.BlockSpec((f_in, f_out), lambda i: (0, 0)),
            pl.BlockSpec((1, f_out), lambda i: (0, 0)),
        ],
        out_specs=pl.BlockSpec((bm_h, f_out), lambda i: (i, 0)),
        out_shape=jax.ShapeDtypeStruct((n, f_out), jnp.bfloat16),
    )(x, W, b.reshape(1, f_out))

    bm = 400
    nsub = 5
    sub = bm // nsub
    adj_specs = [
        pl.BlockSpec((sub, n), lambda i, c=c: (nsub * i + c, 0))
        for c in range(nsub)
    ]
    out = pl.pallas_call(
        _agg_kernel,
        grid=(n // bm,),
        in_specs=adj_specs + [pl.BlockSpec((n, f_out), lambda i: (0, 0))],
        out_specs=pl.BlockSpec((bm, f_out), lambda i: (i, 0)),
        out_shape=jax.ShapeDtypeStruct((n, f_out), jnp.float32),
    )(*([adj] * nsub), h)

    return (out, adj)


# speedup vs baseline: 1.4738x; 1.4738x over previous
import jax
import jax.numpy as jnp
from jax.experimental import pallas as pl


def _h_kernel(x_ref, w_ref, b_ref, h_ref):
    h = jnp.dot(x_ref[...], w_ref[...], preferred_element_type=jnp.float32)
    h_ref[...] = (h + b_ref[...]).astype(jnp.bfloat16)


def _agg_kernel(adj_ref, h_ref, out_ref, adj_out_ref):
    a = adj_ref[...]
    adj_out_ref[...] = a
    acc = jnp.dot(a.astype(jnp.bfloat16), h_ref[...],
                  preferred_element_type=jnp.float32)
    out_ref[...] = jnp.maximum(acc, 0.0)


def kernel(x, adj, W, b):
    n, f_in = x.shape
    f_out = W.shape[1]

    bm_h = 2000
    h = pl.pallas_call(
        _h_kernel,
        grid=(n // bm_h,),
        in_specs=[
            pl.BlockSpec((bm_h, f_in), lambda i: (i, 0)),
            pl.BlockSpec((f_in, f_out), lambda i: (0, 0)),
            pl.BlockSpec((1, f_out), lambda i: (0, 0)),
        ],
        out_specs=pl.BlockSpec((bm_h, f_out), lambda i: (i, 0)),
        out_shape=jax.ShapeDtypeStruct((n, f_out), jnp.bfloat16),
    )(x, W, b.reshape(1, f_out))

    bm = 200
    out, adj_out = pl.pallas_call(
        _agg_kernel,
        grid=(n // bm,),
        in_specs=[
            pl.BlockSpec((bm, n), lambda i: (i, 0)),
            pl.BlockSpec((n, f_out), lambda i: (0, 0)),
        ],
        out_specs=[
            pl.BlockSpec((bm, f_out), lambda i: (i, 0)),
            pl.BlockSpec((bm, n), lambda i: (i, 0)),
        ],
        out_shape=[
            jax.ShapeDtypeStruct((n, f_out), jnp.float32),
            jax.ShapeDtypeStruct((n, n), jnp.float32),
        ],
    )(adj, h)

    return (out, adj_out)


# speedup vs baseline: 1.5046x; 1.0209x over previous
import jax
import jax.numpy as jnp
from jax.experimental import pallas as pl
from jax.experimental.pallas import tpu as pltpu


def _fused_kernel(x_ref, w_ref, b_ref, adj_ref, out_ref, adj_out_ref, h_scr):
    @pl.when(pl.program_id(0) == 0)
    def _():
        h = jnp.dot(x_ref[...], w_ref[...], preferred_element_type=jnp.float32)
        h_scr[...] = (h + b_ref[...]).astype(jnp.bfloat16)

    a = adj_ref[...]
    adj_out_ref[...] = a
    acc = jnp.dot(a.astype(jnp.bfloat16), h_scr[...],
                  preferred_element_type=jnp.float32)
    out_ref[...] = jnp.maximum(acc, 0.0)


def kernel(x, adj, W, b):
    n, f_in = x.shape
    f_out = W.shape[1]
    bm = 200

    out, adj_out = pl.pallas_call(
        _fused_kernel,
        grid=(n // bm,),
        in_specs=[
            pl.BlockSpec((n, f_in), lambda i: (0, 0)),
            pl.BlockSpec((f_in, f_out), lambda i: (0, 0)),
            pl.BlockSpec((1, f_out), lambda i: (0, 0)),
            pl.BlockSpec((bm, n), lambda i: (i, 0)),
        ],
        out_specs=[
            pl.BlockSpec((bm, f_out), lambda i: (i, 0)),
            pl.BlockSpec((bm, n), lambda i: (i, 0)),
        ],
        out_shape=[
            jax.ShapeDtypeStruct((n, f_out), jnp.float32),
            jax.ShapeDtypeStruct((n, n), jnp.float32),
        ],
        scratch_shapes=[pltpu.VMEM((n, f_out), jnp.bfloat16)],
    )(x, W, b.reshape(1, f_out), adj)

    return (out, adj_out)
